# Initial kernel scaffold; baseline (speedup 1.0000x reference)
import jax
import jax.numpy as jnp
from jax.experimental import pallas as pl


def kernel(positions, scale, table):
    raise NotImplementedError("write your pallas kernel here")


# speedup vs baseline: 106.2550x; 106.2550x over previous
import functools

import jax
import jax.numpy as jnp
from jax.experimental import pallas as pl

_S = 2048
_H = 16
_NB = 32
_MAXD = 128
_TI = 16


def _bias_block_kernel(tt_ref, prow_ref, pcol_ref, out_ref):
    prow = prow_ref[0]
    pcol = pcol_ref[...]
    d = prow - pcol
    d = jnp.clip(d, -_MAXD, _MAXD) + _MAXD
    bucket = jnp.minimum(d >> 3, _NB - 1)
    bflat = bucket.reshape(1, _TI * _S)
    iota = jax.lax.broadcasted_iota(jnp.int32, (_NB, _TI * _S), 0)
    onehot = (bflat == iota).astype(jnp.float32)
    res = jnp.dot(tt_ref[...], onehot,
                  preferred_element_type=jnp.float32)
    out_ref[...] = res.reshape(_H, _TI, _S)


@jax.jit
def kernel(positions, scale, table):
    tt = (table * scale[:, 0, 0][None, :]).T
    prows = positions.reshape(_S // _TI, _TI, 1)
    pcols = positions.reshape(1, _S)
    grid = (_S // _TI,)
    out = pl.pallas_call(
        _bias_block_kernel,
        grid=grid,
        in_specs=[
            pl.BlockSpec((_H, _NB), lambda i: (0, 0)),
            pl.BlockSpec((1, _TI, 1), lambda i: (i, 0, 0)),
            pl.BlockSpec((1, _S), lambda i: (0, 0)),
        ],
        out_specs=pl.BlockSpec((_H, _TI, _S), lambda i: (0, i, 0)),
        out_shape=jax.ShapeDtypeStruct((_H, _S, _S), jnp.float32),
    )(tt, prows, pcols)
    return out[None]


# speedup vs baseline: 127.3755x; 1.1988x over previous
import functools

import jax
import jax.numpy as jnp
from jax.experimental import pallas as pl

_S = 2048
_H = 16
_NB = 32
_MAXD = 128
_TI = 32


def _bias_block_kernel(tt_ref, prow_ref, pcol_ref, out_ref):
    prow = prow_ref[0]
    pcol = pcol_ref[...]
    d = prow - pcol
    d = jnp.clip(d, -_MAXD, _MAXD) + _MAXD
    bucket = jnp.minimum(d >> 3, _NB - 1)
    bflat = bucket.reshape(1, _TI * _S)
    iota = jax.lax.broadcasted_iota(jnp.int32, (_NB, _TI * _S), 0)
    onehot = (bflat == iota).astype(jnp.float32)
    res = jnp.dot(tt_ref[...], onehot,
                  preferred_element_type=jnp.float32)
    out_ref[...] = res.reshape(_H, _TI, _S)


@jax.jit
def kernel(positions, scale, table):
    tt = (table * scale[:, 0, 0][None, :]).T
    prows = positions.reshape(_S // _TI, _TI, 1)
    pcols = positions.reshape(1, _S)
    grid = (_S // _TI,)
    out = pl.pallas_call(
        _bias_block_kernel,
        grid=grid,
        in_specs=[
            pl.BlockSpec((_H, _NB), lambda i: (0, 0)),
            pl.BlockSpec((1, _TI, 1), lambda i: (i, 0, 0)),
            pl.BlockSpec((1, _S), lambda i: (0, 0)),
        ],
        out_specs=pl.BlockSpec((_H, _TI, _S), lambda i: (0, i, 0)),
        out_shape=jax.ShapeDtypeStruct((_H, _S, _S), jnp.float32),
    )(tt, prows, pcols)
    return out[None]


# speedup vs baseline: 140.6518x; 1.1042x over previous
import functools

import jax
import jax.numpy as jnp
from jax.experimental import pallas as pl

_S = 2048
_H = 16
_NB = 32
_MAXD = 128
_TI = 64


def _bias_block_kernel(tt_ref, prow_ref, pcol_ref, out_ref):
    prow = prow_ref[0]
    pcol = pcol_ref[...]
    d = prow - pcol
    d = jnp.clip(d, -_MAXD, _MAXD) + _MAXD
    bucket = jnp.minimum(d >> 3, _NB - 1)
    bflat = bucket.reshape(1, _TI * _S)
    iota = jax.lax.broadcasted_iota(jnp.int32, (_NB, _TI * _S), 0)
    onehot = (bflat == iota).astype(jnp.float32)
    res = jnp.dot(tt_ref[...], onehot,
                  preferred_element_type=jnp.float32)
    out_ref[...] = res.reshape(_H, _TI, _S)


@jax.jit
def kernel(positions, scale, table):
    tt = (table * scale[:, 0, 0][None, :]).T
    prows = positions.reshape(_S // _TI, _TI, 1)
    pcols = positions.reshape(1, _S)
    grid = (_S // _TI,)
    out = pl.pallas_call(
        _bias_block_kernel,
        grid=grid,
        in_specs=[
            pl.BlockSpec((_H, _NB), lambda i: (0, 0)),
            pl.BlockSpec((1, _TI, 1), lambda i: (i, 0, 0)),
            pl.BlockSpec((1, _S), lambda i: (0, 0)),
        ],
        out_specs=pl.BlockSpec((_H, _TI, _S), lambda i: (0, i, 0)),
        out_shape=jax.ShapeDtypeStruct((_H, _S, _S), jnp.float32),
    )(tt, prows, pcols)
    return out[None]
